# Initial kernel scaffold; baseline (speedup 1.0000x reference)
import jax
import jax.numpy as jnp
from jax.experimental import pallas as pl


def kernel(x):
    raise NotImplementedError("write your pallas kernel here")


# speedup vs baseline: 3063.3194x; 3063.3194x over previous
import functools
import math

import jax
import jax.numpy as jnp
from jax import lax
from jax.experimental import pallas as pl
from jax.experimental.pallas import tpu as pltpu
from jax.experimental.pallas import tpu_sc as plsc

B = 128
L = 4096
F = L // 2
NW = 32
BINS_W = F // NW
WIN = 384
NG = 6
SCR = 16 * NG
SQ = float(1.0 / math.sqrt(2.0))


def _sc_body(x_hbm, out_hbm, xloc, am, az, ap, aq, ar, rowbuf, outstage):
    wid = lax.axis_index("s") * 2 + lax.axis_index("c")
    cstart = jnp.minimum(jnp.maximum(wid * 128 - 128, 0), L - WIN)
    fstart = cstart // 2
    own_lb0 = wid * BINS_W - fstart
    gbase = jnp.minimum(jnp.maximum(own_lb0 // 16 - 1, 0), WIN // 32 - NG)

    cstart = pl.multiple_of(cstart, 128)
    pltpu.sync_copy(x_hbm.at[:, pl.ds(cstart, WIN)], xloc)

    lane = lax.iota(jnp.int32, 16)

    for g in range(NG):
        ce = lane * 2 + 32 * (gbase + g)
        co = ce + 1

        def fold(j, carry):
            vm, vz, vp, vq, vr = carry
            jv = jnp.full((16,), 0, jnp.int32) + j
            ev = plsc.load_gather(xloc, [jv, ce])
            od = plsc.load_gather(xloc, [jv, co])
            aa = ev + od
            dd = ev - od
            neg_a = plsc.bitcast(aa, jnp.int32) < 0
            neg_d = plsc.bitcast(dd, jnp.int32) < 0
            mcls = jnp.logical_and(neg_a, jnp.logical_not(neg_d))
            pcls = jnp.logical_and(neg_d, jnp.logical_not(neg_a))
            zcls = jnp.logical_not(jnp.logical_xor(neg_a, neg_d))
            vm = jnp.where(mcls, aa, vm)
            vz = jnp.where(zcls, aa, vz)
            vp = jnp.where(pcls, aa, vp)
            vq = jnp.where(jnp.logical_not(pcls), aa, vq)
            vr = jnp.where(jnp.logical_not(mcls), aa, vr)
            return vm, vz, vp, vq, vr

        zero = jnp.zeros((16,), jnp.float32)
        vm, vz, vp, vq, vr = lax.fori_loop(
            0, B, fold, (zero, zero, zero, zero, zero))
        sl = pl.ds(16 * g, 16)
        am[sl] = vm * SQ
        az[sl] = vz * SQ
        ap[sl] = vp * SQ
        aq[sl] = vq * SQ
        ar[sl] = vr * SQ

    j127 = jnp.full((16,), B - 1, jnp.int32)
    for og in range(BINS_W // 16):
        lb = own_lb0 + 16 * og + lane
        slb = lb - 16 * gbase
        binv = wid * BINS_W + 16 * og + lane
        slbm1 = jnp.maximum(slb - 1, 0)
        slbp1 = jnp.minimum(slb + 1, SCR - 1)
        apv = plsc.load_gather(ap, [slbm1])
        azv = plsc.load_gather(az, [slb])
        amv = plsc.load_gather(am, [slbp1])
        add0 = apv + azv + amv
        aqv = plsc.load_gather(aq, [slb])
        arv = plsc.load_gather(ar, [slb])
        add0 = jnp.where(binv == 0, aqv + amv, add0)
        add0 = jnp.where(binv == F - 1, arv + apv, add0)
        ev = plsc.load_gather(xloc, [j127, lb * 2])
        od = plsc.load_gather(xloc, [j127, lb * 2 + 1])
        add1 = (ev - od) * SQ
        idxa = 32 * og + lane * 2
        plsc.store_scatter(rowbuf, [idxa], add0)
        plsc.store_scatter(rowbuf, [idxa + 1], add1)

    rvs = [rowbuf[pl.ds(16 * k, 16)] for k in range(8)]

    def rep(r, carry):
        rv = jnp.full((16,), 0, jnp.int32) + r
        for k in range(8):
            plsc.store_scatter(outstage, [rv, lane + 16 * k], rvs[k])
        return carry

    lax.fori_loop(0, B, rep, 0)
    ostart = pl.multiple_of(wid * 128, 128)
    pltpu.sync_copy(outstage, out_hbm.at[:, pl.ds(ostart, 128)])


@functools.partial(
    pl.kernel,
    mesh=plsc.VectorSubcoreMesh(core_axis_name="c", subcore_axis_name="s"),
    out_type=jax.ShapeDtypeStruct((B, L), jnp.float32),
    compiler_params=pltpu.CompilerParams(needs_layout_passes=False),
    scratch_types=[
        pltpu.VMEM((B, WIN), jnp.float32),
        pltpu.VMEM((SCR,), jnp.float32),
        pltpu.VMEM((SCR,), jnp.float32),
        pltpu.VMEM((SCR,), jnp.float32),
        pltpu.VMEM((SCR,), jnp.float32),
        pltpu.VMEM((SCR,), jnp.float32),
        pltpu.VMEM((128,), jnp.float32),
        pltpu.VMEM((B, 128), jnp.float32),
    ],
)
def _sst_sc(x_hbm, out_hbm, *scratch):
    _sc_body(x_hbm, out_hbm, *scratch)


def kernel(x):
    out2d = _sst_sc(x)
    return out2d.reshape(B, F, 2)
